# Initial kernel scaffold; baseline (speedup 1.0000x reference)
import jax
import jax.numpy as jnp
from jax.experimental import pallas as pl


def kernel(node_features, edge_index, node_pos, edge_attr, params1, params2, fe1, fe2):
    raise NotImplementedError("write your pallas kernel here")


# speedup vs baseline: 2.3691x; 2.3691x over previous
import functools

import jax
import jax.numpy as jnp
from jax import lax
from jax.experimental import pallas as pl
from jax.experimental.pallas import tpu as pltpu
from jax.experimental.pallas import tpu_sc as plsc

NC, NS = 2, 16
NW = NC * NS
CP = 16
SUB = 125
CHUNK = 1000
NSUB = CHUNK // SUB


def _silu(x):
    return x * jax.nn.sigmoid(x)


def _make_gather(E, N, H):
    EW = E // NW
    NIT = EW // CHUNK
    mesh = plsc.VectorSubcoreMesh(core_axis_name="c", subcore_axis_name="s",
                                  num_cores=NC, num_subcores=NS)

    HF = CHUNK // 2
    NSH = NSUB // 2

    @functools.partial(
        pl.kernel, mesh=mesh,
        out_type=[jax.ShapeDtypeStruct((E, H), jnp.float32),
                  jax.ShapeDtypeStruct((E, H), jnp.float32),
                  jax.ShapeDtypeStruct((E, CP), jnp.float32),
                  jax.ShapeDtypeStruct((E, CP), jnp.float32)],
        scratch_types=[pltpu.VMEM((NSUB, SUB), jnp.int32),
                       pltpu.VMEM((NSUB, SUB), jnp.int32),
                       pltpu.VMEM((HF, H), jnp.float32),
                       pltpu.VMEM((HF, CP), jnp.float32),
                       pltpu.SemaphoreType.DMA,
                       pltpu.SemaphoreType.DMA],
        compiler_params=pltpu.CompilerParams(use_tc_tiling_on_sc=False),
    )
    def k(h_hbm, c_hbm, row_hbm, col_hbm,
          hr_out, hc_out, cr_out, cc_out,
          idx_r, idx_c, hbuf, cbuf, s1, s2):
        wid = lax.axis_index("s") * NC + lax.axis_index("c")
        base = wid * EW
        rbase = wid * (EW // SUB)

        def body(i, _):
            r0 = rbase + i * NSUB
            pltpu.sync_copy(row_hbm.at[pl.ds(r0, NSUB)], idx_r)
            pltpu.sync_copy(col_hbm.at[pl.ds(r0, NSUB)], idx_c)
            for half in range(2):
                off = base + i * CHUNK + half * HF
                for idx, h_out, c_out in ((idx_r, hr_out, cr_out),
                                          (idx_c, hc_out, cc_out)):
                    ds = []
                    for j in range(NSH):
                        sl = pl.ds(j * SUB, SUB)
                        jj = half * NSH + j
                        ds.append(pltpu.async_copy(
                            h_hbm.at[idx.at[jj]], hbuf.at[sl], s1))
                        ds.append(pltpu.async_copy(
                            c_hbm.at[idx.at[jj]], cbuf.at[sl], s2))
                    for d in ds:
                        d.wait()
                    pltpu.sync_copy(hbuf, h_out.at[pl.ds(off, HF)])
                    pltpu.sync_copy(cbuf, c_out.at[pl.ds(off, HF)])
            return 0

        lax.fori_loop(0, NIT, body, 0)

    return k


def _make_scatter(E, N, H):
    EC = E // NC
    ET = EC // NS
    NIT = ET // CHUNK
    RT = N // NS
    mesh = plsc.VectorSubcoreMesh(core_axis_name="c", subcore_axis_name="s",
                                  num_cores=NC, num_subcores=NS)

    HF = CHUNK // 2
    NSH = NSUB // 2

    @functools.partial(
        pl.kernel, mesh=mesh,
        out_type=[jax.ShapeDtypeStruct((NC * N, H), jnp.float32),
                  jax.ShapeDtypeStruct((NC * N, CP), jnp.float32)],
        scratch_types=[pltpu.VMEM((NSUB, SUB), jnp.int32),
                       pltpu.VMEM((HF, H), jnp.float32),
                       pltpu.VMEM((HF, CP), jnp.float32),
                       pltpu.VMEM_SHARED((N, H), jnp.float32),
                       pltpu.VMEM_SHARED((N, CP), jnp.float32)],
        compiler_params=pltpu.CompilerParams(use_tc_tiling_on_sc=False),
    )
    def k(ef_hbm, tr_hbm, row_hbm, zh_hbm, zp_hbm,
          agg_out, trs_out, idx, efb, trb, acc_h, acc_p):
        cid = lax.axis_index("c")
        sid = lax.axis_index("s")
        rsl = pl.ds(sid * RT, RT)
        pltpu.sync_copy(zh_hbm.at[rsl], acc_h.at[rsl])
        pltpu.sync_copy(zp_hbm.at[rsl], acc_p.at[rsl])
        plsc.subcore_barrier()
        base = cid * EC + sid * ET
        rbase = cid * (EC // SUB) + sid * (ET // SUB)

        def body(i, _):
            r0 = rbase + i * NSUB
            pltpu.sync_copy(row_hbm.at[pl.ds(r0, NSUB)], idx)
            for half in range(2):
                off = base + i * CHUNK + half * HF
                pltpu.sync_copy(ef_hbm.at[pl.ds(off, HF)], efb)
                pltpu.sync_copy(tr_hbm.at[pl.ds(off, HF)], trb)
                for j in range(NSH):
                    sl = pl.ds(j * SUB, SUB)
                    jj = half * NSH + j
                    pltpu.sync_copy(efb.at[sl], acc_h.at[idx.at[jj]], add=True)
                    pltpu.sync_copy(trb.at[sl], acc_p.at[idx.at[jj]], add=True)
            return 0

        lax.fori_loop(0, NIT, body, 0)
        plsc.subcore_barrier()
        obase = cid * N + sid * RT
        pltpu.sync_copy(acc_h.at[rsl], agg_out.at[pl.ds(obase, RT)])
        pltpu.sync_copy(acc_p.at[rsl], trs_out.at[pl.ds(obase, RT)])

    return k


def _linear(x, w, b):
    M, K = x.shape
    Nn = w.shape[1]

    def body(x_r, w_r, b_r, o_r):
        o_r[...] = (jnp.dot(x_r[...], w_r[...],
                            preferred_element_type=jnp.float32) + b_r[...])

    return pl.pallas_call(
        body, out_shape=jax.ShapeDtypeStruct((M, Nn), jnp.float32),
    )(x, w, b.reshape(1, Nn))


def _edge_call(hr, hc, cr, cc, ea, w0hh, w0r, w0e, b0, w1, b1,
               wa, ba, wc0, bc0, wc1):
    E, H = hr.shape
    EA = ea.shape[1]
    B = 5000
    G = E // B

    def body(hr_r, hc_r, cr_r, cc_r, ea_r, w0hh_r, w0r_r, w0e_r, b0_r,
             w1_r, b1_r, wa_r, ba_r, wc0_r, bc0_r, wc1_r, ef_o, tr_o):
        cd = cr_r[...] - cc_r[...]
        radial = jnp.sum(cd * cd, axis=1, keepdims=True)
        norm = jnp.sqrt(radial) + 1e-8
        hh = jnp.concatenate([hr_r[...], hc_r[...]], axis=1)
        e = (jnp.dot(hh, w0hh_r[...], preferred_element_type=jnp.float32)
             + jnp.dot(ea_r[...], w0e_r[...], preferred_element_type=jnp.float32)
             + radial * w0r_r[...] + b0_r[...])
        e = _silu(e)
        e = _silu(jnp.dot(e, w1_r[...], preferred_element_type=jnp.float32)
                  + b1_r[...])
        att = jax.nn.sigmoid(jnp.sum(e * wa_r[...], axis=1, keepdims=True)
                             + ba_r[...])
        ef = e * att
        mv = _silu(jnp.dot(ef, wc0_r[...], preferred_element_type=jnp.float32)
                   + bc0_r[...])
        m = jnp.sum(mv * wc1_r[...], axis=1, keepdims=True)
        lane = lax.broadcasted_iota(jnp.int32, (1, CP), 1)
        ef_o[...] = ef
        tr_o[...] = (cd / norm) * m + (lane == 3).astype(jnp.float32)

    full = lambda s: pl.BlockSpec(s, lambda i: (0, 0))
    return pl.pallas_call(
        body,
        grid=(G,),
        in_specs=[pl.BlockSpec((B, H), lambda i: (i, 0)),
                  pl.BlockSpec((B, H), lambda i: (i, 0)),
                  pl.BlockSpec((B, CP), lambda i: (i, 0)),
                  pl.BlockSpec((B, CP), lambda i: (i, 0)),
                  pl.BlockSpec((B, EA), lambda i: (i, 0)),
                  full((2 * H, H)), full((1, H)), full((EA, H)), full((1, H)),
                  full((H, H)), full((1, H)), full((1, H)), full((1, 1)),
                  full((H, H)), full((1, H)), full((1, H))],
        out_specs=[pl.BlockSpec((B, H), lambda i: (i, 0)),
                   pl.BlockSpec((B, CP), lambda i: (i, 0))],
        out_shape=[jax.ShapeDtypeStruct((E, H), jnp.float32),
                   jax.ShapeDtypeStruct((E, CP), jnp.float32)],
        compiler_params=pltpu.CompilerParams(
            dimension_semantics=("arbitrary",)),
    )(hr, hc, cr, cc, ea, w0hh, w0r, w0e, b0, w1, b1, wa, ba, wc0, bc0, wc1)


def _node_call(h, aggp, trp, cpad, w0h, w0a, b0, w1, b1):
    N, H = h.shape

    def body(h_r, ap_r, tp_r, c_r, w0h_r, w0a_r, b0_r, w1_r, b1_r, ho, co):
        agg = ap_r[0:N] + ap_r[N:2 * N]
        tr = tp_r[0:N] + tp_r[N:2 * N]
        cnt = jnp.maximum(tr[:, 3:4], 1.0)
        lane = lax.broadcasted_iota(jnp.int32, (1, CP), 1)
        co[...] = c_r[...] + jnp.where(lane < 3, tr / cnt, 0.0)
        o = _silu(jnp.dot(h_r[...], w0h_r[...], preferred_element_type=jnp.float32)
                  + jnp.dot(agg, w0a_r[...], preferred_element_type=jnp.float32)
                  + b0_r[...])
        o = jnp.dot(o, w1_r[...], preferred_element_type=jnp.float32) + b1_r[...]
        ho[...] = h_r[...] + o

    return pl.pallas_call(
        body,
        out_shape=[jax.ShapeDtypeStruct((N, H), jnp.float32),
                   jax.ShapeDtypeStruct((N, CP), jnp.float32)],
    )(h, aggp, trp, cpad, w0h, w0a, b0.reshape(1, H), w1, b1.reshape(1, H))


def _heads_call(h, f1w0, f1b0, f1w1, f1b1, f2w0, f2b0, f2w1, f2b1):
    N, H = h.shape
    PD = f1w0.shape[1]
    OD = f1w1.shape[1]

    def body(h_r, a0, a1, a2, a3, b0_, b1_, b2_, b3_, o1, o2):
        g = jnp.mean(h_r[...], axis=0, keepdims=True)
        t1 = jnp.maximum(jnp.dot(g, a0[...], preferred_element_type=jnp.float32)
                         + a1[...], 0.0)
        o1[...] = jnp.dot(t1, a2[...], preferred_element_type=jnp.float32) + a3[...]
        t2 = jnp.maximum(jnp.dot(g, b0_[...], preferred_element_type=jnp.float32)
                         + b1_[...], 0.0)
        o2[...] = jnp.dot(t2, b2_[...], preferred_element_type=jnp.float32) + b3_[...]

    return pl.pallas_call(
        body,
        out_shape=[jax.ShapeDtypeStruct((1, OD), jnp.float32),
                   jax.ShapeDtypeStruct((1, OD), jnp.float32)],
    )(h,
      f1w0, f1b0.reshape(1, PD), f1w1, f1b1.reshape(1, OD),
      f2w0, f2b0.reshape(1, PD), f2w1, f2b1.reshape(1, OD))


def _run_egnn(params, h, cpad, row2d, col2d, ea, E, N):
    H = params["emb_in"]["w"].shape[1]
    EA = ea.shape[1]
    h = _linear(h, params["emb_in"]["w"], params["emb_in"]["b"])
    gather = _make_gather(E, N, H)
    scatter = _make_scatter(E, N, H)
    zh = jnp.zeros((N, H), jnp.float32)
    zp = jnp.zeros((N, CP), jnp.float32)
    for lp in params["layers"]:
        hr, hc, cr, cc = gather(h, cpad, row2d, col2d)
        w0 = lp["edge_mlp0"]["w"]
        ef, tr = _edge_call(
            hr, hc, cr, cc, ea,
            w0[0:2 * H], w0[2 * H:2 * H + 1], w0[2 * H + 1:2 * H + 1 + EA],
            lp["edge_mlp0"]["b"].reshape(1, H),
            lp["edge_mlp1"]["w"], lp["edge_mlp1"]["b"].reshape(1, H),
            lp["att_mlp"]["w"].reshape(1, H), lp["att_mlp"]["b"].reshape(1, 1),
            lp["coord_mlp0"]["w"], lp["coord_mlp0"]["b"].reshape(1, H),
            lp["coord_mlp1_w"].reshape(1, H))
        aggp, trp = scatter(ef, tr, row2d, zh, zp)
        nw0 = lp["node_mlp0"]["w"]
        h, cpad = _node_call(h, aggp, trp, cpad,
                             nw0[0:H], nw0[H:2 * H], lp["node_mlp0"]["b"],
                             lp["node_mlp1"]["w"], lp["node_mlp1"]["b"])
    h = _linear(h, params["emb_out"]["w"], params["emb_out"]["b"])
    return h, cpad


def kernel(node_features, edge_index, node_pos, edge_attr,
           params1, params2, fe1, fe2):
    N = node_features.shape[0]
    E = edge_index.shape[1]
    row2d = edge_index[0].reshape(E // SUB, SUB)
    col2d = edge_index[1].reshape(E // SUB, SUB)
    cpad = jnp.pad(node_pos, ((0, 0), (0, CP - 3)))
    h, cpad = _run_egnn(params1, node_features, cpad, row2d, col2d,
                        edge_attr, E, N)
    h, cpad = _run_egnn(params2, h, cpad, row2d, col2d, edge_attr, E, N)
    return _heads_call(h,
                       fe1["l0"]["w"], fe1["l0"]["b"], fe1["l1"]["w"], fe1["l1"]["b"],
                       fe2["l0"]["w"], fe2["l0"]["b"], fe2["l1"]["w"], fe2["l1"]["b"])
